# baseline (speedup 1.0000x reference)
import functools

import jax
import jax.numpy as jnp
from jax.experimental import pallas as pl
from jax.experimental.pallas import tpu as pltpu

_HOP = 1920
_D = 512
_K = 2048
_Q = 8
_TILE = 512


def _enc_body(frames_ref, w_ref, b_ref, lat_ref):
    lat = jnp.dot(frames_ref[...].astype(jnp.bfloat16),
                  w_ref[...].astype(jnp.bfloat16),
                  preferred_element_type=jnp.float32)
    lat_ref[...] = lat + b_ref[...]


def _rvq_body(lat_ref, cb_ref, codes_ref,
              res_ref, cbh_ref, cbm_ref, cbl_ref, cn_ref):
    q = pl.program_id(0)
    i = pl.program_id(1)
    rows = pl.ds(i * _TILE, _TILE)

    @pl.when(q == 0)
    def _load_latent():
        res_ref[rows, :] = lat_ref[rows, :]

    @pl.when(i == 0)
    def _prep_codebook():
        cb = cb_ref[0]
        cb_hi = cb.astype(jnp.bfloat16)
        rem = cb - cb_hi.astype(jnp.float32)
        cb_mid = rem.astype(jnp.bfloat16)
        cbh_ref[...] = cb_hi
        cbm_ref[...] = cb_mid
        cbl_ref[...] = (rem - cb_mid.astype(jnp.float32)).astype(jnp.bfloat16)
        cn_ref[...] = jnp.sum(cb * cb, axis=1)[None, :]

    r = res_ref[rows, :]
    rnorm = jnp.sum(r * r, axis=1, keepdims=True)
    scores = jax.lax.dot_general(
        r.astype(jnp.bfloat16), cbh_ref[...], (((1,), (1,)), ((), ())),
        preferred_element_type=jnp.float32)
    d = rnorm - 2.0 * scores + cn_ref[...]

    iota = jax.lax.broadcasted_iota(jnp.int32, (_TILE, _K), 1)
    minv = jnp.min(d, axis=1, keepdims=True)
    idx = jnp.min(jnp.where(d == minv, iota, _K), axis=1)
    codes_ref[0, 0, :] = idx

    onehot = (iota == idx[:, None]).astype(jnp.bfloat16)
    g = lambda part_ref: jax.lax.dot_general(
        onehot, part_ref[...], (((1,), (0,)), ((), ())),
        preferred_element_type=jnp.float32)
    quant = (g(cbh_ref) + g(cbm_ref)) + g(cbl_ref)
    res_ref[rows, :] = r - quant


@functools.partial(jax.jit, static_argnames=())
def kernel(audio_input, W_enc, b_enc, codebooks):
    B = audio_input.shape[0]
    x = audio_input.reshape(B, -1)
    T = x.shape[1] // _HOP
    rows = B * T
    frames = x[:, : T * _HOP].reshape(rows, _HOP)
    n_tiles = (rows + _TILE - 1) // _TILE
    padded = n_tiles * _TILE
    if padded != rows:
        frames = jnp.concatenate(
            [frames, jnp.zeros((padded - rows, _HOP), jnp.float32)], axis=0)

    latent = pl.pallas_call(
        _enc_body,
        grid=(n_tiles,),
        in_specs=[
            pl.BlockSpec((_TILE, _HOP), lambda i: (i, 0)),
            pl.BlockSpec((_HOP, _D), lambda i: (0, 0)),
            pl.BlockSpec((1, _D), lambda i: (0, 0)),
        ],
        out_specs=pl.BlockSpec((_TILE, _D), lambda i: (i, 0)),
        out_shape=jax.ShapeDtypeStruct((padded, _D), jnp.float32),
    )(frames, W_enc, b_enc.reshape(1, _D))

    codes = pl.pallas_call(
        _rvq_body,
        grid=(_Q, n_tiles),
        in_specs=[
            pl.BlockSpec((padded, _D), lambda q, i: (0, 0)),
            pl.BlockSpec((1, _K, _D), lambda q, i: (q, 0, 0)),
        ],
        out_specs=pl.BlockSpec(
            (1, 1, _TILE), lambda q, i, nt=n_tiles: (q * nt + i, 0, 0)),
        out_shape=jax.ShapeDtypeStruct((_Q * n_tiles, 1, _TILE), jnp.int32),
        scratch_shapes=[pltpu.VMEM((padded, _D), jnp.float32),
                        pltpu.VMEM((_K, _D), jnp.bfloat16),
                        pltpu.VMEM((_K, _D), jnp.bfloat16),
                        pltpu.VMEM((_K, _D), jnp.bfloat16),
                        pltpu.VMEM((1, _K), jnp.float32)],
    )(latent, codebooks)

    codes = codes.reshape(_Q, padded)[:, :rows]
    codes = codes.reshape(_Q, B, T).transpose(1, 0, 2)
    return codes.astype(jnp.int32)


# speedup vs baseline: 1.0074x; 1.0074x over previous
import functools

import jax
import jax.numpy as jnp
from jax.experimental import pallas as pl
from jax.experimental.pallas import tpu as pltpu

_HOP = 1920
_D = 512
_K = 2048
_Q = 8
_TILE = 512


def _enc_body(frames_ref, w_ref, b_ref, lat_ref):
    lat = jnp.dot(frames_ref[...].astype(jnp.bfloat16),
                  w_ref[...].astype(jnp.bfloat16),
                  preferred_element_type=jnp.float32)
    lat_ref[...] = lat + b_ref[...]


def _rvq_body(lat_ref, cb_ref, codes_ref,
              res_ref, cbh_ref, cbm_ref, cbl_ref, cn_ref):
    q = pl.program_id(0)
    i = pl.program_id(1)

    @pl.when(q == 0)
    def _load_latent():
        rows2 = pl.ds(i * 2 * _TILE, 2 * _TILE)
        res_ref[rows2, :] = lat_ref[rows2, :]

    @pl.when(i == 0)
    def _prep_codebook():
        cb = cb_ref[0]
        cb_hi = cb.astype(jnp.bfloat16)
        rem = cb - cb_hi.astype(jnp.float32)
        cb_mid = rem.astype(jnp.bfloat16)
        cbh_ref[...] = cb_hi
        cbm_ref[...] = cb_mid
        cbl_ref[...] = (rem - cb_mid.astype(jnp.float32)).astype(jnp.bfloat16)
        cn_ref[...] = jnp.sum(cb * cb, axis=1)[None, :]

    iota = jax.lax.broadcasted_iota(jnp.int32, (_TILE, _K), 1)
    for h in range(2):
        rows = pl.ds((2 * i + h) * _TILE, _TILE)
        r = res_ref[rows, :]
        rnorm = jnp.sum(r * r, axis=1, keepdims=True)
        scores = jax.lax.dot_general(
            r.astype(jnp.bfloat16), cbh_ref[...], (((1,), (1,)), ((), ())),
            preferred_element_type=jnp.float32)
        d = rnorm - 2.0 * scores + cn_ref[...]

        minv = jnp.min(d, axis=1, keepdims=True)
        idx = jnp.min(jnp.where(d == minv, iota, _K), axis=1)
        codes_ref[0, 0, pl.ds(h * _TILE, _TILE)] = idx

        onehot = (iota == idx[:, None]).astype(jnp.bfloat16)
        g = lambda part_ref: jax.lax.dot_general(
            onehot, part_ref[...], (((1,), (0,)), ((), ())),
            preferred_element_type=jnp.float32)
        quant = (g(cbh_ref) + g(cbm_ref)) + g(cbl_ref)
        res_ref[rows, :] = r - quant


@functools.partial(jax.jit, static_argnames=())
def kernel(audio_input, W_enc, b_enc, codebooks):
    B = audio_input.shape[0]
    x = audio_input.reshape(B, -1)
    T = x.shape[1] // _HOP
    rows = B * T
    frames = x[:, : T * _HOP].reshape(rows, _HOP)
    n_tiles = (rows + _TILE - 1) // _TILE
    padded = n_tiles * _TILE
    if padded != rows:
        frames = jnp.concatenate(
            [frames, jnp.zeros((padded - rows, _HOP), jnp.float32)], axis=0)

    latent = pl.pallas_call(
        _enc_body,
        grid=(n_tiles,),
        in_specs=[
            pl.BlockSpec((_TILE, _HOP), lambda i: (i, 0)),
            pl.BlockSpec((_HOP, _D), lambda i: (0, 0)),
            pl.BlockSpec((1, _D), lambda i: (0, 0)),
        ],
        out_specs=pl.BlockSpec((_TILE, _D), lambda i: (i, 0)),
        out_shape=jax.ShapeDtypeStruct((padded, _D), jnp.float32),
    )(frames, W_enc, b_enc.reshape(1, _D))

    n_pairs = n_tiles // 2
    codes = pl.pallas_call(
        _rvq_body,
        grid=(_Q, n_pairs),
        in_specs=[
            pl.BlockSpec((padded, _D), lambda q, i: (0, 0)),
            pl.BlockSpec((1, _K, _D), lambda q, i: (q, 0, 0)),
        ],
        out_specs=pl.BlockSpec(
            (1, 1, 2 * _TILE), lambda q, i, np_=n_pairs: (q * np_ + i, 0, 0)),
        out_shape=jax.ShapeDtypeStruct(
            (_Q * n_pairs, 1, 2 * _TILE), jnp.int32),
        scratch_shapes=[pltpu.VMEM((padded, _D), jnp.float32),
                        pltpu.VMEM((_K, _D), jnp.bfloat16),
                        pltpu.VMEM((_K, _D), jnp.bfloat16),
                        pltpu.VMEM((_K, _D), jnp.bfloat16),
                        pltpu.VMEM((1, _K), jnp.float32)],
    )(latent, codebooks)

    codes = codes.reshape(_Q, padded)[:, :rows]
    codes = codes.reshape(_Q, B, T).transpose(1, 0, 2)
    return codes.astype(jnp.int32)


# speedup vs baseline: 1.0398x; 1.0322x over previous
import functools

import jax
import jax.numpy as jnp
from jax.experimental import pallas as pl
from jax.experimental.pallas import tpu as pltpu

_HOP = 1920
_D = 512
_K = 2048
_Q = 8
_TILE = 512


def _rvq_body(frames_ref, w_ref, b_ref, cb_ref, codes_ref,
              res_ref, cbs_ref, cn_ref):
    q = pl.program_id(0)
    i = pl.program_id(1)
    rows = pl.ds(i * _TILE, _TILE)

    @pl.when(q == 0)
    def _encode():
        lat = jnp.dot(frames_ref[...].astype(jnp.bfloat16),
                      w_ref[...].astype(jnp.bfloat16),
                      preferred_element_type=jnp.float32)
        res_ref[rows, :] = lat + b_ref[...]

    @pl.when(i == 0)
    def _prep_codebook():
        cb = cb_ref[0]
        cb_hi = cb.astype(jnp.bfloat16)
        rem = cb - cb_hi.astype(jnp.float32)
        cb_mid = rem.astype(jnp.bfloat16)
        cbs_ref[:, 0:_D] = cb_hi
        cbs_ref[:, _D:2 * _D] = cb_mid
        cbs_ref[:, 2 * _D:3 * _D] = (
            rem - cb_mid.astype(jnp.float32)).astype(jnp.bfloat16)
        cn_ref[...] = jnp.sum(cb * cb, axis=1)[None, :]

    r = res_ref[rows, :]
    rnorm = jnp.sum(r * r, axis=1, keepdims=True)
    scores = jax.lax.dot_general(
        r.astype(jnp.bfloat16), cbs_ref[:, 0:_D], (((1,), (1,)), ((), ())),
        preferred_element_type=jnp.float32)
    d = rnorm - 2.0 * scores + cn_ref[...]

    idx = jnp.argmin(d, axis=1).astype(jnp.int32)
    codes_ref[0, 0, :] = idx

    iota = jax.lax.broadcasted_iota(jnp.int32, (_TILE, _K), 1)
    onehot = (iota == idx[:, None]).astype(jnp.bfloat16)
    qcat = jax.lax.dot_general(
        onehot, cbs_ref[...], (((1,), (0,)), ((), ())),
        preferred_element_type=jnp.float32)
    quant = ((qcat[:, 0:_D] + qcat[:, _D:2 * _D])
             + qcat[:, 2 * _D:3 * _D])
    res_ref[rows, :] = r - quant


@functools.partial(jax.jit, static_argnames=())
def kernel(audio_input, W_enc, b_enc, codebooks):
    B = audio_input.shape[0]
    x = audio_input.reshape(B, -1)
    T = x.shape[1] // _HOP
    rows = B * T
    frames = x[:, : T * _HOP].reshape(rows, _HOP)
    n_tiles = (rows + _TILE - 1) // _TILE
    padded = n_tiles * _TILE
    if padded != rows:
        frames = jnp.concatenate(
            [frames, jnp.zeros((padded - rows, _HOP), jnp.float32)], axis=0)

    codes = pl.pallas_call(
        _rvq_body,
        grid=(_Q, n_tiles),
        in_specs=[
            pl.BlockSpec((_TILE, _HOP), lambda q, i: (i, 0)),
            pl.BlockSpec((_HOP, _D), lambda q, i: (0, 0)),
            pl.BlockSpec((1, _D), lambda q, i: (0, 0)),
            pl.BlockSpec((1, _K, _D), lambda q, i: (q, 0, 0)),
        ],
        out_specs=pl.BlockSpec(
            (1, 1, _TILE), lambda q, i, nt=n_tiles: (q * nt + i, 0, 0)),
        out_shape=jax.ShapeDtypeStruct((_Q * n_tiles, 1, _TILE), jnp.int32),
        scratch_shapes=[pltpu.VMEM((padded, _D), jnp.float32),
                        pltpu.VMEM((_K, 3 * _D), jnp.bfloat16),
                        pltpu.VMEM((1, _K), jnp.float32)],
    )(frames, W_enc, b_enc.reshape(1, _D), codebooks)

    codes = codes.reshape(_Q, padded)[:, :rows]
    codes = codes.reshape(_Q, B, T).transpose(1, 0, 2)
    return codes.astype(jnp.int32)


# speedup vs baseline: 1.1381x; 1.0945x over previous
import functools

import jax
import jax.numpy as jnp
from jax.experimental import pallas as pl
from jax.experimental.pallas import tpu as pltpu

_HOP = 1920
_D = 512
_K = 2048
_Q = 8
_TILE = 512


def _rvq_body(frames_ref, w_ref, b_ref, cb_ref, codes_ref,
              res_ref, cbs_ref, cn_ref, idxv_ref, idxs_ref, qnt_ref, sem):
    q = pl.program_id(0)
    i = pl.program_id(1)
    rows = pl.ds(i * _TILE, _TILE)

    @pl.when(q == 0)
    def _encode():
        lat = jnp.dot(frames_ref[...].astype(jnp.bfloat16),
                      w_ref[...].astype(jnp.bfloat16),
                      preferred_element_type=jnp.float32)
        res_ref[rows, :] = lat + b_ref[...]

    @pl.when(i == 0)
    def _prep_codebook():
        cb = cb_ref[0]
        cb_hi = cb.astype(jnp.bfloat16)
        rem = cb - cb_hi.astype(jnp.float32)
        cb_mid = rem.astype(jnp.bfloat16)
        cbs_ref[:, 0:_D] = cb_hi
        cbs_ref[:, _D:2 * _D] = cb_mid
        cbs_ref[:, 2 * _D:3 * _D] = (
            rem - cb_mid.astype(jnp.float32)).astype(jnp.bfloat16)
        cn_ref[...] = jnp.sum(cb * cb, axis=1)[None, :]

    r = res_ref[rows, :]
    rnorm = jnp.sum(r * r, axis=1, keepdims=True)
    scores = jax.lax.dot_general(
        r.astype(jnp.bfloat16), cbs_ref[:, 0:_D], (((1,), (1,)), ((), ())),
        preferred_element_type=jnp.float32)
    d = rnorm - 2.0 * scores + cn_ref[...]

    idx = jnp.argmin(d, axis=1).astype(jnp.int32)
    codes_ref[0, 0, :] = idx

    idxv_ref[...] = idx[None, :]
    cp = pltpu.make_async_copy(idxv_ref, idxs_ref, sem)
    cp.start()
    cp.wait()

    def _copy_row(j, _):
        k = idxs_ref[0, j]
        qnt_ref[pl.ds(j, 1), :] = cb_ref[0, pl.ds(k, 1), :]
        return 0

    jax.lax.fori_loop(0, _TILE, _copy_row, 0, unroll=8)
    res_ref[rows, :] = r - qnt_ref[...]


@functools.partial(jax.jit, static_argnames=())
def kernel(audio_input, W_enc, b_enc, codebooks):
    B = audio_input.shape[0]
    x = audio_input.reshape(B, -1)
    T = x.shape[1] // _HOP
    rows = B * T
    frames = x[:, : T * _HOP].reshape(rows, _HOP)
    n_tiles = (rows + _TILE - 1) // _TILE
    padded = n_tiles * _TILE
    if padded != rows:
        frames = jnp.concatenate(
            [frames, jnp.zeros((padded - rows, _HOP), jnp.float32)], axis=0)

    codes = pl.pallas_call(
        _rvq_body,
        grid=(_Q, n_tiles),
        in_specs=[
            pl.BlockSpec((_TILE, _HOP), lambda q, i: (i, 0)),
            pl.BlockSpec((_HOP, _D), lambda q, i: (0, 0)),
            pl.BlockSpec((1, _D), lambda q, i: (0, 0)),
            pl.BlockSpec((1, _K, _D), lambda q, i: (q, 0, 0)),
        ],
        out_specs=pl.BlockSpec(
            (1, 1, _TILE), lambda q, i, nt=n_tiles: (q * nt + i, 0, 0)),
        out_shape=jax.ShapeDtypeStruct((_Q * n_tiles, 1, _TILE), jnp.int32),
        scratch_shapes=[pltpu.VMEM((padded, _D), jnp.float32),
                        pltpu.VMEM((_K, 3 * _D), jnp.bfloat16),
                        pltpu.VMEM((1, _K), jnp.float32),
                        pltpu.VMEM((1, _TILE), jnp.int32),
                        pltpu.SMEM((1, _TILE), jnp.int32),
                        pltpu.VMEM((_TILE, _D), jnp.float32),
                        pltpu.SemaphoreType.DMA],
    )(frames, W_enc, b_enc.reshape(1, _D), codebooks)

    codes = codes.reshape(_Q, padded)[:, :rows]
    codes = codes.reshape(_Q, B, T).transpose(1, 0, 2)
    return codes.astype(jnp.int32)


# speedup vs baseline: 1.1539x; 1.0139x over previous
import functools

import jax
import jax.numpy as jnp
from jax.experimental import pallas as pl
from jax.experimental.pallas import tpu as pltpu

_HOP = 1920
_D = 512
_K = 2048
_Q = 8
_TILE = 512


def _rvq_body(frames_ref, w_ref, b_ref, cb_ref, codes_ref,
              res_ref, cbs_ref, cn_ref, idxv_ref, idxs_ref, qnt_ref, sem):
    q = pl.program_id(0)
    i = pl.program_id(1)
    rows = pl.ds(i * _TILE, _TILE)

    @pl.when(q == 0)
    def _encode():
        lat = jnp.dot(frames_ref[...].astype(jnp.bfloat16),
                      w_ref[...].astype(jnp.bfloat16),
                      preferred_element_type=jnp.float32)
        res_ref[rows, :] = lat + b_ref[...]

    @pl.when(i == 0)
    def _prep_codebook():
        cb = cb_ref[0]
        cb_hi = cb.astype(jnp.bfloat16)
        rem = cb - cb_hi.astype(jnp.float32)
        cb_mid = rem.astype(jnp.bfloat16)
        cbs_ref[:, 0:_D] = cb_hi
        cbs_ref[:, _D:2 * _D] = cb_mid
        cbs_ref[:, 2 * _D:3 * _D] = (
            rem - cb_mid.astype(jnp.float32)).astype(jnp.bfloat16)
        cn_ref[...] = jnp.sum(cb * cb, axis=1)[None, :]

    r = res_ref[rows, :]
    rnorm = jnp.sum(r * r, axis=1, keepdims=True)
    scores = jax.lax.dot_general(
        r.astype(jnp.bfloat16), cbs_ref[:, 0:_D], (((1,), (1,)), ((), ())),
        preferred_element_type=jnp.float32)
    d = rnorm - 2.0 * scores + cn_ref[...]

    idx = jnp.argmin(d, axis=1).astype(jnp.int32)
    codes_ref[0, 0, :] = idx

    idxv_ref[...] = idx[None, :]
    cp = pltpu.make_async_copy(idxv_ref, idxs_ref, sem)
    cp.start()
    cp.wait()

    def _copy_row(j, _):
        k = idxs_ref[0, j]
        qnt_ref[pl.ds(j, 1), :] = cb_ref[0, pl.ds(k, 1), :]
        return 0

    jax.lax.fori_loop(0, _TILE, _copy_row, 0, unroll=32)
    res_ref[rows, :] = r - qnt_ref[...]


@functools.partial(jax.jit, static_argnames=())
def kernel(audio_input, W_enc, b_enc, codebooks):
    B = audio_input.shape[0]
    x = audio_input.reshape(B, -1)
    T = x.shape[1] // _HOP
    rows = B * T
    frames = x[:, : T * _HOP].reshape(rows, _HOP)
    n_tiles = (rows + _TILE - 1) // _TILE
    padded = n_tiles * _TILE
    if padded != rows:
        frames = jnp.concatenate(
            [frames, jnp.zeros((padded - rows, _HOP), jnp.float32)], axis=0)

    codes = pl.pallas_call(
        _rvq_body,
        grid=(_Q, n_tiles),
        in_specs=[
            pl.BlockSpec((_TILE, _HOP), lambda q, i: (i, 0)),
            pl.BlockSpec((_HOP, _D), lambda q, i: (0, 0)),
            pl.BlockSpec((1, _D), lambda q, i: (0, 0)),
            pl.BlockSpec((1, _K, _D), lambda q, i: (q, 0, 0)),
        ],
        out_specs=pl.BlockSpec(
            (1, 1, _TILE), lambda q, i, nt=n_tiles: (q * nt + i, 0, 0)),
        out_shape=jax.ShapeDtypeStruct((_Q * n_tiles, 1, _TILE), jnp.int32),
        scratch_shapes=[pltpu.VMEM((padded, _D), jnp.float32),
                        pltpu.VMEM((_K, 3 * _D), jnp.bfloat16),
                        pltpu.VMEM((1, _K), jnp.float32),
                        pltpu.VMEM((1, _TILE), jnp.int32),
                        pltpu.SMEM((1, _TILE), jnp.int32),
                        pltpu.VMEM((_TILE, _D), jnp.float32),
                        pltpu.SemaphoreType.DMA],
    )(frames, W_enc, b_enc.reshape(1, _D), codebooks)

    codes = codes.reshape(_Q, padded)[:, :rows]
    codes = codes.reshape(_Q, B, T).transpose(1, 0, 2)
    return codes.astype(jnp.int32)


# speedup vs baseline: 1.1779x; 1.0208x over previous
import functools

import jax
import jax.numpy as jnp
from jax.experimental import pallas as pl
from jax.experimental.pallas import tpu as pltpu

_HOP = 1920
_D = 512
_K = 2048
_Q = 8
_TILE = 512


def _rvq_body(frames_ref, w_ref, b_ref, cb_ref, codes_ref,
              res_ref, cbs_ref, cn_ref, idxv_ref, idxs_ref, qnt_ref, sem):
    q = pl.program_id(0)
    i = pl.program_id(1)
    rows = pl.ds(i * _TILE, _TILE)

    @pl.when(q == 0)
    def _encode():
        lat = jnp.dot(frames_ref[...].astype(jnp.bfloat16),
                      w_ref[...].astype(jnp.bfloat16),
                      preferred_element_type=jnp.float32)
        res_ref[rows, :] = lat + b_ref[...]

    @pl.when(i == 0)
    def _prep_codebook():
        cb = cb_ref[0]
        cb_hi = cb.astype(jnp.bfloat16)
        rem = cb - cb_hi.astype(jnp.float32)
        cb_mid = rem.astype(jnp.bfloat16)
        cbs_ref[:, 0:_D] = cb_hi
        cbs_ref[:, _D:2 * _D] = cb_mid
        cbs_ref[:, 2 * _D:3 * _D] = (
            rem - cb_mid.astype(jnp.float32)).astype(jnp.bfloat16)
        cn_ref[...] = jnp.sum(cb * cb, axis=1)[None, :]

    r = res_ref[rows, :]
    rnorm = jnp.sum(r * r, axis=1, keepdims=True)
    scores = jax.lax.dot_general(
        r.astype(jnp.bfloat16), cbs_ref[:, 0:_D], (((1,), (1,)), ((), ())),
        preferred_element_type=jnp.float32)
    d = rnorm - 2.0 * scores + cn_ref[...]

    idx = jnp.argmin(d, axis=1).astype(jnp.int32)
    codes_ref[0, 0, :] = idx

    @pl.when(q < _Q - 1)
    def _gather_update():
        idxv_ref[...] = idx[None, :]
        cp = pltpu.make_async_copy(idxv_ref, idxs_ref, sem)
        cp.start()
        cp.wait()

        def _copy_row(j, _):
            k = idxs_ref[0, j]
            qnt_ref[pl.ds(j, 1), :] = cb_ref[0, pl.ds(k, 1), :]
            return 0

        jax.lax.fori_loop(0, _TILE, _copy_row, 0, unroll=32)
        res_ref[rows, :] = r - qnt_ref[...]


@functools.partial(jax.jit, static_argnames=())
def kernel(audio_input, W_enc, b_enc, codebooks):
    B = audio_input.shape[0]
    x = audio_input.reshape(B, -1)
    T = x.shape[1] // _HOP
    rows = B * T
    frames = x[:, : T * _HOP].reshape(rows, _HOP)
    n_tiles = (rows + _TILE - 1) // _TILE
    padded = n_tiles * _TILE
    if padded != rows:
        frames = jnp.concatenate(
            [frames, jnp.zeros((padded - rows, _HOP), jnp.float32)], axis=0)

    codes = pl.pallas_call(
        _rvq_body,
        grid=(_Q, n_tiles),
        in_specs=[
            pl.BlockSpec((_TILE, _HOP), lambda q, i: (i, 0)),
            pl.BlockSpec((_HOP, _D), lambda q, i: (0, 0)),
            pl.BlockSpec((1, _D), lambda q, i: (0, 0)),
            pl.BlockSpec((1, _K, _D), lambda q, i: (q, 0, 0)),
        ],
        out_specs=pl.BlockSpec(
            (1, 1, _TILE), lambda q, i, nt=n_tiles: (q * nt + i, 0, 0)),
        out_shape=jax.ShapeDtypeStruct((_Q * n_tiles, 1, _TILE), jnp.int32),
        scratch_shapes=[pltpu.VMEM((padded, _D), jnp.float32),
                        pltpu.VMEM((_K, 3 * _D), jnp.bfloat16),
                        pltpu.VMEM((1, _K), jnp.float32),
                        pltpu.VMEM((1, _TILE), jnp.int32),
                        pltpu.SMEM((1, _TILE), jnp.int32),
                        pltpu.VMEM((_TILE, _D), jnp.float32),
                        pltpu.SemaphoreType.DMA],
    )(frames, W_enc, b_enc.reshape(1, _D), codebooks)

    codes = codes.reshape(_Q, padded)[:, :rows]
    codes = codes.reshape(_Q, B, T).transpose(1, 0, 2)
    return codes.astype(jnp.int32)
